# baseline (speedup 1.0000x reference)
import functools

import jax
import jax.numpy as jnp
from jax import lax
from jax.experimental import pallas as pl
from jax.experimental.pallas import tpu as pltpu
from jax.experimental.pallas import tpu_sc as plsc

N = 50000
E = 800000
H = 64
TW = 5
STEPS = 3

NC, NS = 2, 16
NW = NC * NS
CH = 128
HALF = 25088
N_PAD = 2 * HALF
QUART = N_PAD // 4
SPM_ROWS = QUART + 128
E_PAD = 802816
EPW = E_PAD // NW
BLK_E = 1024
BLK_N = 512


def _mm(a, b):
    return jnp.dot(a, b, precision=jax.lax.Precision.HIGHEST)


def _ln(y, g, b):
    mu = jnp.mean(y, axis=-1, keepdims=True)
    var = jnp.mean((y - mu) ** 2, axis=-1, keepdims=True)
    return (y - mu) / jnp.sqrt(var + 1e-5) * g + b


def _encode_body(x_ref, g_ref, w1, b1, w2, b2, lg, lb, wij, vlin,
                 xh_ref, t0_ref):
    x = x_ref[...]
    h = jax.nn.relu(_mm(x, w1[...]) + b1[...])
    xh = _ln(_mm(h, w2[...]) + b2[...], lg[...], lb[...])
    xh_ref[...] = xh
    g = g_ref[...]
    q = _mm(g[:, :3], vlin[...])
    t0_ref[...] = jnp.concatenate(
        [_mm(xh, wij[...]), q, g, jnp.zeros((x.shape[0], 48), jnp.float32)],
        axis=1)


def _encode_call(x0, geom, w1, b1, w2, b2, lg, lb, wij, vlin):
    nb = N_PAD // BLK_N
    row = lambda i: (i, 0)
    full = lambda i: (0, 0)
    return pl.pallas_call(
        _encode_body,
        grid=(nb,),
        in_specs=[
            pl.BlockSpec((BLK_N, 9), row),
            pl.BlockSpec((BLK_N, 16), row),
            pl.BlockSpec((9, H), full),
            pl.BlockSpec((1, H), full),
            pl.BlockSpec((H, H), full),
            pl.BlockSpec((1, H), full),
            pl.BlockSpec((1, H), full),
            pl.BlockSpec((1, H), full),
            pl.BlockSpec((H, 2 * H), full),
            pl.BlockSpec((3, H), full),
        ],
        out_specs=[
            pl.BlockSpec((BLK_N, H), row),
            pl.BlockSpec((BLK_N, 256), row),
        ],
        out_shape=[
            jax.ShapeDtypeStruct((N_PAD, H), jnp.float32),
            jax.ShapeDtypeStruct((N_PAD, 256), jnp.float32),
        ],
        compiler_params=pltpu.CompilerParams(
            dimension_semantics=("parallel",)),
    )(x0, geom, w1, b1, w2, b2, lg, lb, wij, vlin)


GCH = 64
IDXB = 512


def _sc_gather_body(in_w, out_w, t_hbm, s_hbm, r_hbm, out_hbm,
                    idx_s, idx_r, rows_s0, rows_r0, rows_s1, rows_r1,
                    out0, out1, sem_g0, sem_g1, sem_w0, sem_w1):
    wid = lax.axis_index("s") * NC + lax.axis_index("c")
    base = wid * EPW
    rows = ((rows_s0, rows_r0, out0, sem_g0, sem_w0),
            (rows_s1, rows_r1, out1, sem_g1, sem_w1))

    def combine(rs, rr, ov):
        def row(j, c):
            for cc in range(4):
                sa = pl.ds(cc * 16, 16)
                sb = pl.ds(H + cc * 16, 16)
                ov[j, sa] = rr[j, sa] + rs[j, sb]
                ov[j, sb] = rs[j, sa] + rr[j, sb]
            for cc in range(8, out_w // 16):
                sg = pl.ds(cc * 16, 16)
                ov[j, sg] = rs[j, sg] - rr[j, sg]
            return c

        lax.fori_loop(0, GCH, row, 0)

    def block(kb, carry):
        boff = base + kb * IDXB
        pltpu.sync_copy(s_hbm.at[pl.ds(boff, IDXB)], idx_s)
        pltpu.sync_copy(r_hbm.at[pl.ds(boff, IDXB)], idx_r)

        def pair(p, c):
            cps = []
            for b, (rs, rr, ov, sg, sw) in enumerate(rows):
                isl = idx_s.at[pl.ds((2 * p + b) * GCH, GCH)]
                irl = idx_r.at[pl.ds((2 * p + b) * GCH, GCH)]
                cps.append((pltpu.async_copy(t_hbm.at[isl], rs, sg),
                            pltpu.async_copy(t_hbm.at[irl], rr, sg)))
            wbs = []
            for b, (rs, rr, ov, sg, sw) in enumerate(rows):
                cps[b][0].wait()
                cps[b][1].wait()
                combine(rs, rr, ov)
                off = boff + (2 * p + b) * GCH
                wbs.append(pltpu.async_copy(
                    ov, out_hbm.at[pl.ds(off, GCH)], sw))
            for wb in wbs:
                wb.wait()
            return c

        lax.fori_loop(0, IDXB // (2 * GCH), pair, 0)
        return carry

    lax.fori_loop(0, EPW // IDXB, block, 0)


def _make_sc_gather(in_w, out_w):
    mesh = plsc.VectorSubcoreMesh(core_axis_name="c", subcore_axis_name="s")
    return functools.partial(
        pl.kernel,
        out_type=jax.ShapeDtypeStruct((E_PAD, out_w), jnp.float32),
        mesh=mesh,
        scratch_types=[
            pltpu.VMEM((IDXB,), jnp.int32),
            pltpu.VMEM((IDXB,), jnp.int32),
            pltpu.VMEM((GCH, in_w), jnp.float32),
            pltpu.VMEM((GCH, in_w), jnp.float32),
            pltpu.VMEM((GCH, in_w), jnp.float32),
            pltpu.VMEM((GCH, in_w), jnp.float32),
            pltpu.VMEM((GCH, out_w), jnp.float32),
            pltpu.VMEM((GCH, out_w), jnp.float32),
            pltpu.SemaphoreType.DMA,
            pltpu.SemaphoreType.DMA,
            pltpu.SemaphoreType.DMA,
            pltpu.SemaphoreType.DMA,
        ],
    )(functools.partial(_sc_gather_body, in_w, out_w))


ZCH = 264
OCH = 392


def _sc_scatter_body(pass_id, msg_hbm, r_hbm, out_hbm,
                     idx_raw, idx_loc, msg_v, stage_v, aggr_s, sem):
    cid = lax.axis_index("c")
    sid = lax.axis_index("s")

    def zrow(j, c):
        for g in range(H // 16):
            stage_v[j, pl.ds(g * 16, 16)] = jnp.zeros((16,), jnp.float32)
        return c

    lax.fori_loop(0, ZCH, zrow, 0)
    zbase = sid * (SPM_ROWS // NS)
    for c in range(SPM_ROWS // NS // ZCH):
        pltpu.sync_copy(stage_v.at[pl.ds(0, ZCH)],
                        aggr_s.at[pl.ds(zbase + c * ZCH, ZCH)])
    plsc.subcore_barrier()

    node_base = (2 * pass_id + cid) * QUART
    lane = lax.iota(jnp.int32, 16)

    def chunk(k, carry):
        off = (k * NS + sid) * CH
        pltpu.sync_copy(r_hbm.at[pl.ds(off, CH)], idx_raw)
        pltpu.sync_copy(msg_hbm.at[pl.ds(off, CH)], msg_v)
        for g in range(CH // 16):
            sl = pl.ds(g * 16, 16)
            loc = idx_raw[sl] - node_base
            ok = (loc >= 0) & (loc < QUART)
            idx_loc[sl] = jnp.where(ok, loc, QUART + g * 16 + lane)
        pltpu.sync_copy(msg_v, aggr_s.at[idx_loc], add=True)
        return carry

    lax.fori_loop(0, E_PAD // CH // NS, chunk, 0)
    plsc.subcore_barrier()

    obase = sid * (QUART // NS)
    for c in range(QUART // NS // OCH):
        pltpu.sync_copy(aggr_s.at[pl.ds(obase + c * OCH, OCH)], stage_v)
        pltpu.sync_copy(
            stage_v, out_hbm.at[pl.ds(cid * QUART + obase + c * OCH, OCH)])


def _make_sc_scatter(pass_id):
    mesh = plsc.VectorSubcoreMesh(core_axis_name="c", subcore_axis_name="s")
    return functools.partial(
        pl.kernel,
        out_type=jax.ShapeDtypeStruct((2 * QUART, H), jnp.float32),
        mesh=mesh,
        scratch_types=[
            pltpu.VMEM((CH,), jnp.int32),
            pltpu.VMEM((CH,), jnp.int32),
            pltpu.VMEM((CH, H), jnp.float32),
            pltpu.VMEM((OCH, H), jnp.float32),
            pltpu.VMEM_SHARED((SPM_ROWS, H), jnp.float32),
            pltpu.SemaphoreType.DMA,
        ],
        compiler_params=pltpu.CompilerParams(use_tc_tiling_on_sc=False),
    )(functools.partial(_sc_scatter_body, pass_id))


def _edge_mlp(pre_m, pre_n, w2, b2, lg, lb):
    msg = _ln(_mm(jax.nn.relu(pre_m), w2) + b2, lg, lb)
    new_e = _ln(_mm(jax.nn.relu(pre_n), w2) + b2, lg, lb)
    return msg, new_e


def _edge0_body(p_ref, vrow2, c1, v2, c2, vg, vb, w1e, b1e, w2e, b2e, eg, eb,
                msg_ref, ehn_ref):
    p = p_ref[...]
    dx = p[:, 192:193]
    dy = p[:, 193:194]
    dist = jnp.sqrt(dx * dx + dy * dy + 1e-12)
    pre_e = p[:, 128:192] + dist * vrow2[...] + c1[...]
    eh = _ln(_mm(jax.nn.relu(pre_e), v2[...]) + c2[...], vg[...], vb[...])
    t = _mm(eh, w1e[...]) + b1e[...]
    msg, new_e = _edge_mlp(p[:, :H] + t, p[:, H:2 * H] + t,
                           w2e[...], b2e[...], eg[...], eb[...])
    msg_ref[...] = msg
    ehn_ref[...] = eh + new_e


def _edge0_call(part, vrow2, c1, v2, c2, vg, vb, w1e, b1e, w2e, b2e, eg, eb):
    nb = E_PAD // BLK_E
    row = lambda i: (i, 0)
    full = lambda i: (0, 0)
    wspec = [pl.BlockSpec((1, H), full), pl.BlockSpec((1, H), full),
             pl.BlockSpec((H, H), full), pl.BlockSpec((1, H), full),
             pl.BlockSpec((1, H), full), pl.BlockSpec((1, H), full),
             pl.BlockSpec((H, H), full), pl.BlockSpec((1, H), full),
             pl.BlockSpec((H, H), full), pl.BlockSpec((1, H), full),
             pl.BlockSpec((1, H), full), pl.BlockSpec((1, H), full)]
    return pl.pallas_call(
        _edge0_body,
        grid=(nb,),
        in_specs=[pl.BlockSpec((BLK_E, 208), row)] + wspec,
        out_specs=[pl.BlockSpec((BLK_E, H), row),
                   pl.BlockSpec((BLK_E, H), row)],
        out_shape=[jax.ShapeDtypeStruct((E_PAD, H), jnp.float32),
                   jax.ShapeDtypeStruct((E_PAD, H), jnp.float32)],
        compiler_params=pltpu.CompilerParams(
            dimension_semantics=("parallel",)),
    )(part, vrow2, c1, v2, c2, vg, vb, w1e, b1e, w2e, b2e, eg, eb)


def _edge_body(write_eh, p_ref, eh_ref, w1e, b1e, w2e, b2e, eg, eb,
               msg_ref, *maybe_ehn):
    p = p_ref[...]
    eh = eh_ref[...]
    t = _mm(eh, w1e[...]) + b1e[...]
    msg, new_e = _edge_mlp(p[:, :H] + t, p[:, H:] + t,
                           w2e[...], b2e[...], eg[...], eb[...])
    msg_ref[...] = msg
    if write_eh:
        maybe_ehn[0][...] = eh + new_e


def _edge_call(part, eh, w1e, b1e, w2e, b2e, eg, eb, write_eh):
    nb = E_PAD // BLK_E
    row = lambda i: (i, 0)
    full = lambda i: (0, 0)
    wspec = [pl.BlockSpec((H, H), full), pl.BlockSpec((1, H), full),
             pl.BlockSpec((H, H), full), pl.BlockSpec((1, H), full),
             pl.BlockSpec((1, H), full), pl.BlockSpec((1, H), full)]
    out_specs = [pl.BlockSpec((BLK_E, H), row)]
    out_shape = [jax.ShapeDtypeStruct((E_PAD, H), jnp.float32)]
    if write_eh:
        out_specs.append(pl.BlockSpec((BLK_E, H), row))
        out_shape.append(jax.ShapeDtypeStruct((E_PAD, H), jnp.float32))
    res = pl.pallas_call(
        functools.partial(_edge_body, write_eh),
        grid=(nb,),
        in_specs=[pl.BlockSpec((BLK_E, 2 * H), row),
                  pl.BlockSpec((BLK_E, H), row)] + wspec,
        out_specs=out_specs,
        out_shape=out_shape,
        compiler_params=pltpu.CompilerParams(
            dimension_semantics=("parallel",)),
    )(part, eh, w1e, b1e, w2e, b2e, eg, eb)
    return res if write_eh else (res[0], None)


def _node_body(aggr_ref, xh_ref, u1, d1, u2, d2, ng, nb_, wnext,
               xh_out, aux_out):
    cat = jnp.concatenate([aggr_ref[...], xh_ref[...]], axis=1)
    h = jax.nn.relu(_mm(cat, u1[...]) + d1[...])
    xh2 = xh_ref[...] + _ln(_mm(h, u2[...]) + d2[...], ng[...], nb_[...])
    xh_out[...] = xh2
    aux_out[...] = _mm(xh2, wnext[...])


def _node_call(aggr, xh, u1, d1, u2, d2, ng, nb_, wnext, aux_w):
    nblk = N_PAD // BLK_N
    row = lambda i: (i, 0)
    full = lambda i: (0, 0)
    return pl.pallas_call(
        _node_body,
        grid=(nblk,),
        in_specs=[
            pl.BlockSpec((BLK_N, H), row),
            pl.BlockSpec((BLK_N, H), row),
            pl.BlockSpec((2 * H, H), full),
            pl.BlockSpec((1, H), full),
            pl.BlockSpec((H, H), full),
            pl.BlockSpec((1, H), full),
            pl.BlockSpec((1, H), full),
            pl.BlockSpec((1, H), full),
            pl.BlockSpec((H, aux_w), full),
        ],
        out_specs=[pl.BlockSpec((BLK_N, H), row),
                   pl.BlockSpec((BLK_N, aux_w), row)],
        out_shape=[jax.ShapeDtypeStruct((N_PAD, H), jnp.float32),
                   jax.ShapeDtypeStruct((N_PAD, aux_w), jnp.float32)],
        compiler_params=pltpu.CompilerParams(
            dimension_semantics=("parallel",)),
    )(aggr, xh, u1, d1, u2, d2, ng, nb_, wnext)


def _node_final_body(aggr_ref, xh_ref, u1, d1, u2, d2, ng, nb_,
                     dw1, db1, dw2, db2, out_ref):
    cat = jnp.concatenate([aggr_ref[...], xh_ref[...]], axis=1)
    h = jax.nn.relu(_mm(cat, u1[...]) + d1[...])
    xh2 = xh_ref[...] + _ln(_mm(h, u2[...]) + d2[...], ng[...], nb_[...])
    h1 = _mm(xh2, dw1[...]) + db1[...]
    h1 = h1 * jax.nn.sigmoid(h1)
    out_ref[...] = _mm(h1, dw2[...]) + db2[...]


def _node_final_call(aggr, xh, u1, d1, u2, d2, ng, nb_, dw1, db1, dw2, db2):
    nblk = N_PAD // BLK_N
    row = lambda i: (i, 0)
    full = lambda i: (0, 0)
    return pl.pallas_call(
        _node_final_body,
        grid=(nblk,),
        in_specs=[
            pl.BlockSpec((BLK_N, H), row),
            pl.BlockSpec((BLK_N, H), row),
            pl.BlockSpec((2 * H, H), full),
            pl.BlockSpec((1, H), full),
            pl.BlockSpec((H, H), full),
            pl.BlockSpec((1, H), full),
            pl.BlockSpec((1, H), full),
            pl.BlockSpec((1, H), full),
            pl.BlockSpec((H, H // 2), full),
            pl.BlockSpec((1, H // 2), full),
            pl.BlockSpec((H // 2, TW), full),
            pl.BlockSpec((1, TW), full),
        ],
        out_specs=[pl.BlockSpec((BLK_N, TW), row)],
        out_shape=[jax.ShapeDtypeStruct((N_PAD, TW), jnp.float32)],
        compiler_params=pltpu.CompilerParams(
            dimension_semantics=("parallel",)),
    )(aggr, xh, u1, d1, u2, d2, ng, nb_, dw1, db1, dw2, db2)[0]


def _r(v):
    return v.reshape(1, -1)


def kernel(params, temperature, temperature_prev, heat_source,
           heat_source_prev, heat_source_next, mesh_pos, edge_index):
    f32 = jnp.float32
    senders = jnp.pad(edge_index[0], (0, E_PAD - E), constant_values=N)
    receivers = jnp.pad(edge_index[1], (0, E_PAD - E), constant_values=N)

    x0 = jnp.concatenate(
        [temperature[:, None], temperature_prev[:, None],
         heat_source[:, None], heat_source_prev[:, None],
         heat_source_next], axis=1)
    x0 = jnp.pad(x0, ((0, N_PAD - N), (0, 0)))
    geom = jnp.concatenate([mesh_pos, temperature[:, None]], axis=1)
    geom = jnp.pad(geom, ((0, N_PAD - N), (0, 13)))

    ne = params["node_enc"]
    w1 = ne["W1"]
    w1n = jnp.concatenate(
        [(w1[0] + w1[1])[None], (-w1[1])[None],
         (w1[2] + w1[3])[None], (-w1[3])[None], w1[4:9]], axis=0)

    procs = params["procs"]
    wij = [jnp.concatenate([p["edge_mlp"]["W1"][:H],
                            p["edge_mlp"]["W1"][H:2 * H]], axis=1)
           for p in procs]
    w1e = [p["edge_mlp"]["W1"][2 * H:] for p in procs]

    ee = params["edge_enc"]
    vlin = ee["W1"][jnp.array([0, 1, 3])]

    xh, table = _encode_call(
        x0, geom, w1n, _r(ne["b1"]), ne["W2"], _r(ne["b2"]),
        _r(ne["ln_g"]), _r(ne["ln_b"]), wij[0], vlin)

    gather0 = _make_sc_gather(256, 208)
    gather128 = _make_sc_gather(128, 128)
    scatter_lo = _make_sc_scatter(0)
    scatter_hi = _make_sc_scatter(1)
    dec = params["dec"]
    dt = jnp.arange(1, TW + 1, dtype=f32)

    eh = None
    for s in range(STEPS):
        em = procs[s]["edge_mlp"]
        nm = procs[s]["node_mlp"]
        if s == 0:
            part = gather0(table, senders, receivers)
            msg, eh = _edge0_call(
                part, ee["W1"][2:3], _r(ee["b1"]), ee["W2"], _r(ee["b2"]),
                _r(ee["ln_g"]), _r(ee["ln_b"]),
                w1e[0], _r(em["b1"]), em["W2"], _r(em["b2"]),
                _r(em["ln_g"]), _r(em["ln_b"]))
        else:
            part = gather128(table, senders, receivers)
            msg, eh = _edge_call(
                part, eh, w1e[s], _r(em["b1"]), em["W2"], _r(em["b2"]),
                _r(em["ln_g"]), _r(em["ln_b"]), write_eh=(s < STEPS - 1))
        aggr = jnp.concatenate(
            [scatter_lo(msg, receivers), scatter_hi(msg, receivers)], axis=0)
        if s < STEPS - 1:
            xh, table = _node_call(
                aggr, xh, nm["W1"], _r(nm["b1"]), nm["W2"], _r(nm["b2"]),
                _r(nm["ln_g"]), _r(nm["ln_b"]), wij[s + 1], 2 * H)
        else:
            out = _node_final_call(
                aggr, xh, nm["W1"], _r(nm["b1"]), nm["W2"], _r(nm["b2"]),
                _r(nm["ln_g"]), _r(nm["ln_b"]),
                dec["W1"], _r(dec["b1"]), dec["W2"] * dt[None, :],
                _r(dec["b2"] * dt))
    return out[:N]


# speedup vs baseline: 1.2095x; 1.2095x over previous
import functools

import jax
import jax.numpy as jnp
from jax import lax
from jax.experimental import pallas as pl
from jax.experimental.pallas import tpu as pltpu
from jax.experimental.pallas import tpu_sc as plsc

N = 50000
E = 800000
H = 64
TW = 5
STEPS = 3

NC, NS = 2, 16
NW = NC * NS
CH = 128
HALF = 25088
N_PAD = 2 * HALF
QUART = N_PAD // 4
SPM_ROWS = QUART + 128
E_PAD = 802816
EPW = E_PAD // NW
BLK_E = 1024
BLK_N = 512


def _mm(a, b):
    return jnp.dot(a, b, precision=jax.lax.Precision.HIGHEST)


def _ln(y, g, b):
    mu = jnp.mean(y, axis=-1, keepdims=True)
    var = jnp.mean((y - mu) ** 2, axis=-1, keepdims=True)
    return (y - mu) / jnp.sqrt(var + 1e-5) * g + b


def _encode_body(x_ref, g_ref, w1, b1, w2, b2, lg, lb, wij, vlin,
                 xh_ref, t0_ref):
    x = x_ref[...]
    h = jax.nn.relu(_mm(x, w1[...]) + b1[...])
    xh = _ln(_mm(h, w2[...]) + b2[...], lg[...], lb[...])
    xh_ref[...] = xh
    g = g_ref[...]
    q = _mm(g[:, :3], vlin[...])
    t0_ref[...] = jnp.concatenate(
        [_mm(xh, wij[...]), q, g, jnp.zeros((x.shape[0], 48), jnp.float32)],
        axis=1)


def _encode_call(x0, geom, w1, b1, w2, b2, lg, lb, wij, vlin):
    nb = N_PAD // BLK_N
    row = lambda i: (i, 0)
    full = lambda i: (0, 0)
    return pl.pallas_call(
        _encode_body,
        grid=(nb,),
        in_specs=[
            pl.BlockSpec((BLK_N, 9), row),
            pl.BlockSpec((BLK_N, 16), row),
            pl.BlockSpec((9, H), full),
            pl.BlockSpec((1, H), full),
            pl.BlockSpec((H, H), full),
            pl.BlockSpec((1, H), full),
            pl.BlockSpec((1, H), full),
            pl.BlockSpec((1, H), full),
            pl.BlockSpec((H, 2 * H), full),
            pl.BlockSpec((3, H), full),
        ],
        out_specs=[
            pl.BlockSpec((BLK_N, H), row),
            pl.BlockSpec((BLK_N, 256), row),
        ],
        out_shape=[
            jax.ShapeDtypeStruct((N_PAD, H), jnp.float32),
            jax.ShapeDtypeStruct((N_PAD, 256), jnp.float32),
        ],
        compiler_params=pltpu.CompilerParams(
            dimension_semantics=("parallel",)),
    )(x0, geom, w1, b1, w2, b2, lg, lb, wij, vlin)


GCH = 64
IDXB = 512


def _sc_gather_body(in_w, out_w, t_hbm, s_hbm, r_hbm, out_hbm,
                    idx_s, idx_r, rows_s0, rows_r0, rows_s1, rows_r1,
                    out0, out1, sem_g0, sem_g1, sem_w0, sem_w1):
    wid = lax.axis_index("s") * NC + lax.axis_index("c")
    base = wid * EPW
    rows = ((rows_s0, rows_r0, out0, sem_g0, sem_w0),
            (rows_s1, rows_r1, out1, sem_g1, sem_w1))

    def combine(rs, rr, ov):
        def row(j, c):
            for cc in range(4):
                sa = pl.ds(cc * 16, 16)
                sb = pl.ds(H + cc * 16, 16)
                ov[j, sa] = rr[j, sa] + rs[j, sb]
                ov[j, sb] = rs[j, sa] + rr[j, sb]
            for cc in range(8, out_w // 16):
                sg = pl.ds(cc * 16, 16)
                ov[j, sg] = rs[j, sg] - rr[j, sg]
            return c

        lax.fori_loop(0, GCH, row, 0)

    def block(kb, carry):
        boff = base + kb * IDXB
        pltpu.sync_copy(s_hbm.at[pl.ds(boff, IDXB)], idx_s)
        pltpu.sync_copy(r_hbm.at[pl.ds(boff, IDXB)], idx_r)

        def pair(p, c):
            cps = []
            for b, (rs, rr, ov, sg, sw) in enumerate(rows):
                isl = idx_s.at[pl.ds((2 * p + b) * GCH, GCH)]
                irl = idx_r.at[pl.ds((2 * p + b) * GCH, GCH)]
                cps.append((pltpu.async_copy(t_hbm.at[isl], rs, sg),
                            pltpu.async_copy(t_hbm.at[irl], rr, sg)))
            wbs = []
            for b, (rs, rr, ov, sg, sw) in enumerate(rows):
                cps[b][0].wait()
                cps[b][1].wait()
                combine(rs, rr, ov)
                off = boff + (2 * p + b) * GCH
                wbs.append(pltpu.async_copy(
                    ov, out_hbm.at[pl.ds(off, GCH)], sw))
            for wb in wbs:
                wb.wait()
            return c

        lax.fori_loop(0, IDXB // (2 * GCH), pair, 0)
        return carry

    lax.fori_loop(0, EPW // IDXB, block, 0)


def _make_sc_gather(in_w, out_w):
    mesh = plsc.VectorSubcoreMesh(core_axis_name="c", subcore_axis_name="s")
    return functools.partial(
        pl.kernel,
        out_type=jax.ShapeDtypeStruct((E_PAD, out_w), jnp.float32),
        mesh=mesh,
        scratch_types=[
            pltpu.VMEM((IDXB,), jnp.int32),
            pltpu.VMEM((IDXB,), jnp.int32),
            pltpu.VMEM((GCH, in_w), jnp.float32),
            pltpu.VMEM((GCH, in_w), jnp.float32),
            pltpu.VMEM((GCH, in_w), jnp.float32),
            pltpu.VMEM((GCH, in_w), jnp.float32),
            pltpu.VMEM((GCH, out_w), jnp.float32),
            pltpu.VMEM((GCH, out_w), jnp.float32),
            pltpu.SemaphoreType.DMA,
            pltpu.SemaphoreType.DMA,
            pltpu.SemaphoreType.DMA,
            pltpu.SemaphoreType.DMA,
        ],
    )(functools.partial(_sc_gather_body, in_w, out_w))


ZCH = 264
OCH = 392


def _sc_scatter_body(pass_id, msg_hbm, r_hbm, out_hbm,
                     idx_raw, idx_loc, msg_v, stage_v, aggr_s, sem):
    cid = lax.axis_index("c")
    sid = lax.axis_index("s")

    def zrow(j, c):
        for g in range(H // 16):
            stage_v[j, pl.ds(g * 16, 16)] = jnp.zeros((16,), jnp.float32)
        return c

    lax.fori_loop(0, ZCH, zrow, 0)
    zbase = sid * (SPM_ROWS // NS)
    for c in range(SPM_ROWS // NS // ZCH):
        pltpu.sync_copy(stage_v.at[pl.ds(0, ZCH)],
                        aggr_s.at[pl.ds(zbase + c * ZCH, ZCH)])
    plsc.subcore_barrier()

    node_base = (2 * pass_id + cid) * QUART
    lane = lax.iota(jnp.int32, 16)

    def chunk(k, carry):
        off = (k * NS + sid) * CH
        pltpu.sync_copy(r_hbm.at[pl.ds(off, CH)], idx_raw)
        pltpu.sync_copy(msg_hbm.at[pl.ds(off, CH)], msg_v)
        for g in range(CH // 16):
            sl = pl.ds(g * 16, 16)
            loc = idx_raw[sl] - node_base
            ok = (loc >= 0) & (loc < QUART)
            idx_loc[sl] = jnp.where(ok, loc, QUART + g * 16 + lane)
        pltpu.sync_copy(msg_v, aggr_s.at[idx_loc], add=True)
        return carry

    lax.fori_loop(0, E_PAD // CH // NS, chunk, 0)
    plsc.subcore_barrier()

    obase = sid * (QUART // NS)
    for c in range(QUART // NS // OCH):
        pltpu.sync_copy(aggr_s.at[pl.ds(obase + c * OCH, OCH)], stage_v)
        pltpu.sync_copy(
            stage_v, out_hbm.at[pl.ds(cid * QUART + obase + c * OCH, OCH)])


def _make_sc_scatter(pass_id):
    mesh = plsc.VectorSubcoreMesh(core_axis_name="c", subcore_axis_name="s")
    return functools.partial(
        pl.kernel,
        out_type=jax.ShapeDtypeStruct((2 * QUART, H), jnp.float32),
        mesh=mesh,
        scratch_types=[
            pltpu.VMEM((CH,), jnp.int32),
            pltpu.VMEM((CH,), jnp.int32),
            pltpu.VMEM((CH, H), jnp.float32),
            pltpu.VMEM((OCH, H), jnp.float32),
            pltpu.VMEM_SHARED((SPM_ROWS, H), jnp.float32),
            pltpu.SemaphoreType.DMA,
        ],
        compiler_params=pltpu.CompilerParams(use_tc_tiling_on_sc=False),
    )(functools.partial(_sc_scatter_body, pass_id))


def _edge_mlp(pre_m, pre_n, w2, b2, lg, lb):
    msg = _ln(jax.nn.relu(pre_m) @ w2 + b2, lg, lb)
    new_e = _ln(jax.nn.relu(pre_n) @ w2 + b2, lg, lb)
    return msg, new_e


def _edge0_body(p_ref, vrow2, c1, v2, c2, vg, vb, w1e, b1e, w2e, b2e, eg, eb,
                msg_ref, ehn_ref):
    p = p_ref[...]
    dx = p[:, 192:193]
    dy = p[:, 193:194]
    dist = jnp.sqrt(dx * dx + dy * dy + 1e-12)
    pre_e = p[:, 128:192] + dist * vrow2[...] + c1[...]
    eh = _ln(jax.nn.relu(pre_e) @ v2[...] + c2[...], vg[...], vb[...])
    t = eh @ w1e[...] + b1e[...]
    msg, new_e = _edge_mlp(p[:, :H] + t, p[:, H:2 * H] + t,
                           w2e[...], b2e[...], eg[...], eb[...])
    msg_ref[...] = msg
    ehn_ref[...] = eh + new_e


def _edge0_call(part, vrow2, c1, v2, c2, vg, vb, w1e, b1e, w2e, b2e, eg, eb):
    nb = E_PAD // BLK_E
    row = lambda i: (i, 0)
    full = lambda i: (0, 0)
    wspec = [pl.BlockSpec((1, H), full), pl.BlockSpec((1, H), full),
             pl.BlockSpec((H, H), full), pl.BlockSpec((1, H), full),
             pl.BlockSpec((1, H), full), pl.BlockSpec((1, H), full),
             pl.BlockSpec((H, H), full), pl.BlockSpec((1, H), full),
             pl.BlockSpec((H, H), full), pl.BlockSpec((1, H), full),
             pl.BlockSpec((1, H), full), pl.BlockSpec((1, H), full)]
    return pl.pallas_call(
        _edge0_body,
        grid=(nb,),
        in_specs=[pl.BlockSpec((BLK_E, 208), row)] + wspec,
        out_specs=[pl.BlockSpec((BLK_E, H), row),
                   pl.BlockSpec((BLK_E, H), row)],
        out_shape=[jax.ShapeDtypeStruct((E_PAD, H), jnp.float32),
                   jax.ShapeDtypeStruct((E_PAD, H), jnp.float32)],
        compiler_params=pltpu.CompilerParams(
            dimension_semantics=("parallel",)),
    )(part, vrow2, c1, v2, c2, vg, vb, w1e, b1e, w2e, b2e, eg, eb)


def _edge_body(write_eh, p_ref, eh_ref, w1e, b1e, w2e, b2e, eg, eb,
               msg_ref, *maybe_ehn):
    p = p_ref[...]
    eh = eh_ref[...]
    t = eh @ w1e[...] + b1e[...]
    msg, new_e = _edge_mlp(p[:, :H] + t, p[:, H:] + t,
                           w2e[...], b2e[...], eg[...], eb[...])
    msg_ref[...] = msg
    if write_eh:
        maybe_ehn[0][...] = eh + new_e


def _edge_call(part, eh, w1e, b1e, w2e, b2e, eg, eb, write_eh):
    nb = E_PAD // BLK_E
    row = lambda i: (i, 0)
    full = lambda i: (0, 0)
    wspec = [pl.BlockSpec((H, H), full), pl.BlockSpec((1, H), full),
             pl.BlockSpec((H, H), full), pl.BlockSpec((1, H), full),
             pl.BlockSpec((1, H), full), pl.BlockSpec((1, H), full)]
    out_specs = [pl.BlockSpec((BLK_E, H), row)]
    out_shape = [jax.ShapeDtypeStruct((E_PAD, H), jnp.float32)]
    if write_eh:
        out_specs.append(pl.BlockSpec((BLK_E, H), row))
        out_shape.append(jax.ShapeDtypeStruct((E_PAD, H), jnp.float32))
    res = pl.pallas_call(
        functools.partial(_edge_body, write_eh),
        grid=(nb,),
        in_specs=[pl.BlockSpec((BLK_E, 2 * H), row),
                  pl.BlockSpec((BLK_E, H), row)] + wspec,
        out_specs=out_specs,
        out_shape=out_shape,
        compiler_params=pltpu.CompilerParams(
            dimension_semantics=("parallel",)),
    )(part, eh, w1e, b1e, w2e, b2e, eg, eb)
    return res if write_eh else (res[0], None)


def _node_body(aggr_ref, xh_ref, u1, d1, u2, d2, ng, nb_, wnext,
               xh_out, aux_out):
    cat = jnp.concatenate([aggr_ref[...], xh_ref[...]], axis=1)
    h = jax.nn.relu(_mm(cat, u1[...]) + d1[...])
    xh2 = xh_ref[...] + _ln(_mm(h, u2[...]) + d2[...], ng[...], nb_[...])
    xh_out[...] = xh2
    aux_out[...] = _mm(xh2, wnext[...])


def _node_call(aggr, xh, u1, d1, u2, d2, ng, nb_, wnext, aux_w):
    nblk = N_PAD // BLK_N
    row = lambda i: (i, 0)
    full = lambda i: (0, 0)
    return pl.pallas_call(
        _node_body,
        grid=(nblk,),
        in_specs=[
            pl.BlockSpec((BLK_N, H), row),
            pl.BlockSpec((BLK_N, H), row),
            pl.BlockSpec((2 * H, H), full),
            pl.BlockSpec((1, H), full),
            pl.BlockSpec((H, H), full),
            pl.BlockSpec((1, H), full),
            pl.BlockSpec((1, H), full),
            pl.BlockSpec((1, H), full),
            pl.BlockSpec((H, aux_w), full),
        ],
        out_specs=[pl.BlockSpec((BLK_N, H), row),
                   pl.BlockSpec((BLK_N, aux_w), row)],
        out_shape=[jax.ShapeDtypeStruct((N_PAD, H), jnp.float32),
                   jax.ShapeDtypeStruct((N_PAD, aux_w), jnp.float32)],
        compiler_params=pltpu.CompilerParams(
            dimension_semantics=("parallel",)),
    )(aggr, xh, u1, d1, u2, d2, ng, nb_, wnext)


def _node_final_body(aggr_ref, xh_ref, u1, d1, u2, d2, ng, nb_,
                     dw1, db1, dw2, db2, out_ref):
    cat = jnp.concatenate([aggr_ref[...], xh_ref[...]], axis=1)
    h = jax.nn.relu(_mm(cat, u1[...]) + d1[...])
    xh2 = xh_ref[...] + _ln(_mm(h, u2[...]) + d2[...], ng[...], nb_[...])
    h1 = _mm(xh2, dw1[...]) + db1[...]
    h1 = h1 * jax.nn.sigmoid(h1)
    out_ref[...] = _mm(h1, dw2[...]) + db2[...]


def _node_final_call(aggr, xh, u1, d1, u2, d2, ng, nb_, dw1, db1, dw2, db2):
    nblk = N_PAD // BLK_N
    row = lambda i: (i, 0)
    full = lambda i: (0, 0)
    return pl.pallas_call(
        _node_final_body,
        grid=(nblk,),
        in_specs=[
            pl.BlockSpec((BLK_N, H), row),
            pl.BlockSpec((BLK_N, H), row),
            pl.BlockSpec((2 * H, H), full),
            pl.BlockSpec((1, H), full),
            pl.BlockSpec((H, H), full),
            pl.BlockSpec((1, H), full),
            pl.BlockSpec((1, H), full),
            pl.BlockSpec((1, H), full),
            pl.BlockSpec((H, H // 2), full),
            pl.BlockSpec((1, H // 2), full),
            pl.BlockSpec((H // 2, TW), full),
            pl.BlockSpec((1, TW), full),
        ],
        out_specs=[pl.BlockSpec((BLK_N, TW), row)],
        out_shape=[jax.ShapeDtypeStruct((N_PAD, TW), jnp.float32)],
        compiler_params=pltpu.CompilerParams(
            dimension_semantics=("parallel",)),
    )(aggr, xh, u1, d1, u2, d2, ng, nb_, dw1, db1, dw2, db2)[0]


def _r(v):
    return v.reshape(1, -1)


def kernel(params, temperature, temperature_prev, heat_source,
           heat_source_prev, heat_source_next, mesh_pos, edge_index):
    f32 = jnp.float32
    senders = jnp.pad(edge_index[0], (0, E_PAD - E), constant_values=N)
    receivers = jnp.pad(edge_index[1], (0, E_PAD - E), constant_values=N)

    x0 = jnp.concatenate(
        [temperature[:, None], temperature_prev[:, None],
         heat_source[:, None], heat_source_prev[:, None],
         heat_source_next], axis=1)
    x0 = jnp.pad(x0, ((0, N_PAD - N), (0, 0)))
    geom = jnp.concatenate([mesh_pos, temperature[:, None]], axis=1)
    geom = jnp.pad(geom, ((0, N_PAD - N), (0, 13)))

    ne = params["node_enc"]
    w1 = ne["W1"]
    w1n = jnp.concatenate(
        [(w1[0] + w1[1])[None], (-w1[1])[None],
         (w1[2] + w1[3])[None], (-w1[3])[None], w1[4:9]], axis=0)

    procs = params["procs"]
    wij = [jnp.concatenate([p["edge_mlp"]["W1"][:H],
                            p["edge_mlp"]["W1"][H:2 * H]], axis=1)
           for p in procs]
    w1e = [p["edge_mlp"]["W1"][2 * H:] for p in procs]

    ee = params["edge_enc"]
    vlin = ee["W1"][jnp.array([0, 1, 3])]

    xh, table = _encode_call(
        x0, geom, w1n, _r(ne["b1"]), ne["W2"], _r(ne["b2"]),
        _r(ne["ln_g"]), _r(ne["ln_b"]), wij[0], vlin)

    gather0 = _make_sc_gather(256, 208)
    gather128 = _make_sc_gather(128, 128)
    scatter_lo = _make_sc_scatter(0)
    scatter_hi = _make_sc_scatter(1)
    dec = params["dec"]
    dt = jnp.arange(1, TW + 1, dtype=f32)

    eh = None
    for s in range(STEPS):
        em = procs[s]["edge_mlp"]
        nm = procs[s]["node_mlp"]
        if s == 0:
            part = gather0(table, senders, receivers)
            msg, eh = _edge0_call(
                part, ee["W1"][2:3], _r(ee["b1"]), ee["W2"], _r(ee["b2"]),
                _r(ee["ln_g"]), _r(ee["ln_b"]),
                w1e[0], _r(em["b1"]), em["W2"], _r(em["b2"]),
                _r(em["ln_g"]), _r(em["ln_b"]))
        else:
            part = gather128(table, senders, receivers)
            msg, eh = _edge_call(
                part, eh, w1e[s], _r(em["b1"]), em["W2"], _r(em["b2"]),
                _r(em["ln_g"]), _r(em["ln_b"]), write_eh=(s < STEPS - 1))
        aggr = jnp.concatenate(
            [scatter_lo(msg, receivers), scatter_hi(msg, receivers)], axis=0)
        if s < STEPS - 1:
            xh, table = _node_call(
                aggr, xh, nm["W1"], _r(nm["b1"]), nm["W2"], _r(nm["b2"]),
                _r(nm["ln_g"]), _r(nm["ln_b"]), wij[s + 1], 2 * H)
        else:
            out = _node_final_call(
                aggr, xh, nm["W1"], _r(nm["b1"]), nm["W2"], _r(nm["b2"]),
                _r(nm["ln_g"]), _r(nm["ln_b"]),
                dec["W1"], _r(dec["b1"]), dec["W2"] * dt[None, :],
                _r(dec["b2"] * dt))
    return out[:N]
